# baseline (device time: 32972 ns/iter reference)
import jax
import jax.numpy as jnp
from jax import lax
from jax.experimental import pallas as pl
from jax.experimental.pallas import tpu as pltpu


def kernel(A, B):
    M, K = A.shape
    K2, N = B.shape
    assert K == K2

    def body(a_ref, b_ref, out_ref, comm_ref, send_sem, recv_sem):
        my_x = lax.axis_index("x")
        my_y = lax.axis_index("y")
        peer = (1 - my_x, my_y)

        partial = jnp.dot(
            a_ref[...].astype(jnp.bfloat16),
            b_ref[...].astype(jnp.bfloat16),
            preferred_element_type=jnp.float32,
        )
        comm_ref[0, :, :] = partial.astype(jnp.bfloat16)

        barrier_sem = pltpu.get_barrier_semaphore()
        pl.semaphore_signal(
            barrier_sem, inc=1,
            device_id=peer, device_id_type=pl.DeviceIdType.MESH,
        )
        pl.semaphore_wait(barrier_sem, 1)

        rdma = pltpu.make_async_remote_copy(
            src_ref=comm_ref.at[0],
            dst_ref=comm_ref.at[1],
            send_sem=send_sem,
            recv_sem=recv_sem,
            device_id=peer,
            device_id_type=pl.DeviceIdType.MESH,
        )
        rdma.start()
        rdma.wait()

        out_ref[...] = partial + comm_ref[1, :, :].astype(jnp.float32)

    return pl.pallas_call(
        body,
        out_shape=jax.ShapeDtypeStruct((M, N), jnp.float32),
        in_specs=[
            pl.BlockSpec(memory_space=pltpu.VMEM),
            pl.BlockSpec(memory_space=pltpu.VMEM),
        ],
        out_specs=pl.BlockSpec(memory_space=pltpu.VMEM),
        scratch_shapes=[
            pltpu.VMEM((2, M, N), jnp.bfloat16),
            pltpu.SemaphoreType.DMA,
            pltpu.SemaphoreType.DMA,
        ],
        compiler_params=pltpu.CompilerParams(collective_id=0),
    )(A, B)


# device time: 25814 ns/iter; 1.2773x vs baseline; 1.2773x over previous
import jax
import jax.numpy as jnp
from jax import lax
from jax.experimental import pallas as pl
from jax.experimental.pallas import tpu as pltpu

C = 4


def kernel(A, B):
    M, K = A.shape
    K2, N = B.shape
    assert K == K2
    Mh = M // 2
    Rc = Mh // C

    def body(a_ref, b_ref, out_ref,
             xsend, xrecv, ysend, yrecv,
             xsend_sems, xrecv_sems, ysend_sems, yrecv_sems):
        my_x = lax.axis_index("x")
        my_y = lax.axis_index("y")
        peer_x = (1 - my_x, my_y)
        peer_y = (my_x, 1 - my_y)
        my_base = my_y * Mh
        other_base = (1 - my_y) * Mh

        barrier_sem = pltpu.get_barrier_semaphore()
        for peer in (peer_x, peer_y):
            pl.semaphore_signal(
                barrier_sem, inc=1,
                device_id=peer, device_id_type=pl.DeviceIdType.MESH,
            )
        pl.semaphore_wait(barrier_sem, 2)

        b_bf16 = b_ref[...].astype(jnp.bfloat16)

        x_rdmas = []
        for c in range(C):
            a_c = a_ref[pl.ds(my_base + c * Rc, Rc), :].astype(jnp.bfloat16)
            part_c = jnp.dot(a_c, b_bf16, preferred_element_type=jnp.float32)
            xsend[pl.ds(c * Rc, Rc), :] = part_c.astype(jnp.bfloat16)
            rdma = pltpu.make_async_remote_copy(
                src_ref=xsend.at[pl.ds(c * Rc, Rc), :],
                dst_ref=xrecv.at[pl.ds(c * Rc, Rc), :],
                send_sem=xsend_sems.at[c],
                recv_sem=xrecv_sems.at[c],
                device_id=peer_x,
                device_id_type=pl.DeviceIdType.MESH,
            )
            rdma.start()
            x_rdmas.append(rdma)

        y_rdmas = []
        for c in range(C):
            x_rdmas[c].wait_recv()
            red_c = (xsend[pl.ds(c * Rc, Rc), :].astype(jnp.float32)
                     + xrecv[pl.ds(c * Rc, Rc), :].astype(jnp.float32))
            out_ref[pl.ds(my_base + c * Rc, Rc), :] = red_c
            ysend[pl.ds(c * Rc, Rc), :] = red_c.astype(jnp.bfloat16)
            rdma = pltpu.make_async_remote_copy(
                src_ref=ysend.at[pl.ds(c * Rc, Rc), :],
                dst_ref=yrecv.at[pl.ds(c * Rc, Rc), :],
                send_sem=ysend_sems.at[c],
                recv_sem=yrecv_sems.at[c],
                device_id=peer_y,
                device_id_type=pl.DeviceIdType.MESH,
            )
            rdma.start()
            y_rdmas.append(rdma)

        for c in range(C):
            y_rdmas[c].wait_recv()
            out_ref[pl.ds(other_base + c * Rc, Rc), :] = (
                yrecv[pl.ds(c * Rc, Rc), :].astype(jnp.float32))

        for c in range(C):
            x_rdmas[c].wait_send()
            y_rdmas[c].wait_send()

    return pl.pallas_call(
        body,
        out_shape=jax.ShapeDtypeStruct((M, N), jnp.float32),
        in_specs=[
            pl.BlockSpec(memory_space=pltpu.VMEM),
            pl.BlockSpec(memory_space=pltpu.VMEM),
        ],
        out_specs=pl.BlockSpec(memory_space=pltpu.VMEM),
        scratch_shapes=[
            pltpu.VMEM((Mh, N), jnp.bfloat16),
            pltpu.VMEM((Mh, N), jnp.bfloat16),
            pltpu.VMEM((Mh, N), jnp.bfloat16),
            pltpu.VMEM((Mh, N), jnp.bfloat16),
            pltpu.SemaphoreType.DMA((C,)),
            pltpu.SemaphoreType.DMA((C,)),
            pltpu.SemaphoreType.DMA((C,)),
            pltpu.SemaphoreType.DMA((C,)),
        ],
        compiler_params=pltpu.CompilerParams(collective_id=0),
    )(A, B)


# device time: 25121 ns/iter; 1.3125x vs baseline; 1.0276x over previous
import jax
import jax.numpy as jnp
from jax import lax
from jax.experimental import pallas as pl
from jax.experimental.pallas import tpu as pltpu

C = 4


def kernel(A, B):
    M, K = A.shape
    K2, N = B.shape
    assert K == K2
    Mh = M // 2
    Rc = Mh // C

    def body(a_ref, b_ref, out_ref,
             xsend, xrecv,
             xsend_sems, xrecv_sems, ysend_sems, yrecv_sems):
        my_x = lax.axis_index("x")
        my_y = lax.axis_index("y")
        peer_x = (1 - my_x, my_y)
        peer_y = (my_x, 1 - my_y)
        my_base = my_y * Mh

        barrier_sem = pltpu.get_barrier_semaphore()
        for peer in (peer_x, peer_y):
            pl.semaphore_signal(
                barrier_sem, inc=1,
                device_id=peer, device_id_type=pl.DeviceIdType.MESH,
            )
        pl.semaphore_wait(barrier_sem, 2)

        b_bf16 = b_ref[...].astype(jnp.bfloat16)

        x_rdmas = []
        for c in range(C):
            a_c = a_ref[pl.ds(my_base + c * Rc, Rc), :].astype(jnp.bfloat16)
            part_c = jnp.dot(a_c, b_bf16, preferred_element_type=jnp.float32)
            xsend[pl.ds(c * Rc, Rc), :] = part_c.astype(jnp.bfloat16)
            rdma = pltpu.make_async_remote_copy(
                src_ref=xsend.at[pl.ds(c * Rc, Rc), :],
                dst_ref=xrecv.at[pl.ds(c * Rc, Rc), :],
                send_sem=xsend_sems.at[c],
                recv_sem=xrecv_sems.at[c],
                device_id=peer_x,
                device_id_type=pl.DeviceIdType.MESH,
            )
            rdma.start()
            x_rdmas.append(rdma)

        y_rdmas = []
        for c in range(C):
            x_rdmas[c].wait_recv()
            red_c = (xsend[pl.ds(c * Rc, Rc), :].astype(jnp.float32)
                     + xrecv[pl.ds(c * Rc, Rc), :].astype(jnp.float32))
            out_ref[pl.ds(my_base + c * Rc, Rc), :] = red_c.astype(jnp.bfloat16)
            rdma = pltpu.make_async_remote_copy(
                src_ref=out_ref.at[pl.ds(my_base + c * Rc, Rc), :],
                dst_ref=out_ref.at[pl.ds(my_base + c * Rc, Rc), :],
                send_sem=ysend_sems.at[c],
                recv_sem=yrecv_sems.at[c],
                device_id=peer_y,
                device_id_type=pl.DeviceIdType.MESH,
            )
            rdma.start()
            y_rdmas.append(rdma)

        for c in range(C):
            y_rdmas[c].wait_recv()
        for c in range(C):
            x_rdmas[c].wait_send()
            y_rdmas[c].wait_send()

    return pl.pallas_call(
        body,
        out_shape=jax.ShapeDtypeStruct((M, N), jnp.bfloat16),
        in_specs=[
            pl.BlockSpec(memory_space=pltpu.VMEM),
            pl.BlockSpec(memory_space=pltpu.VMEM),
        ],
        out_specs=pl.BlockSpec(memory_space=pltpu.VMEM),
        scratch_shapes=[
            pltpu.VMEM((Mh, N), jnp.bfloat16),
            pltpu.VMEM((Mh, N), jnp.bfloat16),
            pltpu.SemaphoreType.DMA((C,)),
            pltpu.SemaphoreType.DMA((C,)),
            pltpu.SemaphoreType.DMA((C,)),
            pltpu.SemaphoreType.DMA((C,)),
        ],
        compiler_params=pltpu.CompilerParams(collective_id=0),
    )(A, B)


# device time: 24367 ns/iter; 1.3531x vs baseline; 1.0309x over previous
import jax
import jax.numpy as jnp
from jax import lax
from jax.experimental import pallas as pl
from jax.experimental.pallas import tpu as pltpu

C = 8


def kernel(A, B):
    M, K = A.shape
    K2, N = B.shape
    assert K == K2
    Mh = M // 2
    Rc = Mh // C

    def body(a_hbm, b_hbm, out_hbm,
             a_v, b_v, xsend, xrecv, ystage, yrecv,
             in_sems, out_sems,
             xsend_sems, xrecv_sems, ysend_sems, yrecv_sems):
        my_x = lax.axis_index("x")
        my_y = lax.axis_index("y")
        peer_x = (1 - my_x, my_y)
        peer_y = (my_x, 1 - my_y)
        my_base = my_y * Mh
        other_base = (1 - my_y) * Mh

        cp_b = pltpu.make_async_copy(b_hbm, b_v, in_sems.at[0])
        cp_b.start()
        cp_a = pltpu.make_async_copy(
            a_hbm.at[pl.ds(my_base, Mh), :], a_v, in_sems.at[1])
        cp_a.start()

        barrier_sem = pltpu.get_barrier_semaphore()
        for peer in (peer_x, peer_y):
            pl.semaphore_signal(
                barrier_sem, inc=1,
                device_id=peer, device_id_type=pl.DeviceIdType.MESH,
            )
        pl.semaphore_wait(barrier_sem, 2)
        cp_b.wait()
        cp_a.wait()

        b_bf16 = b_v[...].astype(jnp.bfloat16)

        x_rdmas = []
        for c in range(C):
            a_c = a_v[pl.ds(c * Rc, Rc), :].astype(jnp.bfloat16)
            part_c = jnp.dot(a_c, b_bf16, preferred_element_type=jnp.float32)
            xsend[pl.ds(c * Rc, Rc), :] = part_c.astype(jnp.bfloat16)
            rdma = pltpu.make_async_remote_copy(
                src_ref=xsend.at[pl.ds(c * Rc, Rc), :],
                dst_ref=xrecv.at[pl.ds(c * Rc, Rc), :],
                send_sem=xsend_sems.at[c],
                recv_sem=xrecv_sems.at[c],
                device_id=peer_x,
                device_id_type=pl.DeviceIdType.MESH,
            )
            rdma.start()
            x_rdmas.append(rdma)

        y_rdmas = []
        out_cps = []
        for c in range(C):
            x_rdmas[c].wait_recv()
            red_c = (xsend[pl.ds(c * Rc, Rc), :].astype(jnp.float32)
                     + xrecv[pl.ds(c * Rc, Rc), :].astype(jnp.float32))
            ystage[pl.ds(c * Rc, Rc), :] = red_c.astype(jnp.bfloat16)
            rdma = pltpu.make_async_remote_copy(
                src_ref=ystage.at[pl.ds(c * Rc, Rc), :],
                dst_ref=yrecv.at[pl.ds(c * Rc, Rc), :],
                send_sem=ysend_sems.at[c],
                recv_sem=yrecv_sems.at[c],
                device_id=peer_y,
                device_id_type=pl.DeviceIdType.MESH,
            )
            rdma.start()
            y_rdmas.append(rdma)
            cp = pltpu.make_async_copy(
                ystage.at[pl.ds(c * Rc, Rc), :],
                out_hbm.at[pl.ds(my_base + c * Rc, Rc), :],
                out_sems.at[c])
            cp.start()
            out_cps.append(cp)

        for c in range(C):
            y_rdmas[c].wait_recv()
            cp = pltpu.make_async_copy(
                yrecv.at[pl.ds(c * Rc, Rc), :],
                out_hbm.at[pl.ds(other_base + c * Rc, Rc), :],
                out_sems.at[C + c])
            cp.start()
            out_cps.append(cp)

        for cp in out_cps:
            cp.wait()
        for c in range(C):
            x_rdmas[c].wait_send()
            y_rdmas[c].wait_send()

    return pl.pallas_call(
        body,
        out_shape=jax.ShapeDtypeStruct((M, N), jnp.bfloat16),
        in_specs=[
            pl.BlockSpec(memory_space=pl.ANY),
            pl.BlockSpec(memory_space=pl.ANY),
        ],
        out_specs=pl.BlockSpec(memory_space=pl.ANY),
        scratch_shapes=[
            pltpu.VMEM((Mh, K), jnp.float32),
            pltpu.VMEM((K, N), jnp.float32),
            pltpu.VMEM((Mh, N), jnp.bfloat16),
            pltpu.VMEM((Mh, N), jnp.bfloat16),
            pltpu.VMEM((Mh, N), jnp.bfloat16),
            pltpu.VMEM((Mh, N), jnp.bfloat16),
            pltpu.SemaphoreType.DMA((2,)),
            pltpu.SemaphoreType.DMA((2 * C,)),
            pltpu.SemaphoreType.DMA((C,)),
            pltpu.SemaphoreType.DMA((C,)),
            pltpu.SemaphoreType.DMA((C,)),
            pltpu.SemaphoreType.DMA((C,)),
        ],
        compiler_params=pltpu.CompilerParams(collective_id=0),
    )(A, B)


# device time: 24097 ns/iter; 1.3683x vs baseline; 1.0112x over previous
import jax
import jax.numpy as jnp
from jax import lax
from jax.experimental import pallas as pl
from jax.experimental.pallas import tpu as pltpu

C = 8


def kernel(A, B):
    M, K = A.shape
    K2, N = B.shape
    assert K == K2
    Mh = M // 2
    Rc = Mh // C

    def body(a_hbm, b_hbm, out_hbm,
             a_v, b_v, xsend, xrecv, ystage, yrecv,
             in_sems, out_sems,
             xsend_sems, xrecv_sems, ysend_sems, yrecv_sems):
        my_x = lax.axis_index("x")
        my_y = lax.axis_index("y")
        peer_x = (1 - my_x, my_y)
        peer_y = (my_x, 1 - my_y)
        my_base = my_y * Mh
        other_base = (1 - my_y) * Mh

        barrier_sem = pltpu.get_barrier_semaphore()
        for peer in (peer_x, peer_y):
            pl.semaphore_signal(
                barrier_sem, inc=1,
                device_id=peer, device_id_type=pl.DeviceIdType.MESH,
            )
        cp_b = pltpu.make_async_copy(b_hbm, b_v, in_sems.at[0])
        cp_b.start()
        cp_a = pltpu.make_async_copy(
            a_hbm.at[pl.ds(my_base, Mh), :], a_v, in_sems.at[1])
        cp_a.start()
        cp_b.wait()
        cp_a.wait()

        b_bf16 = b_v[...].astype(jnp.bfloat16)

        x_rdmas = []
        for c in range(C):
            a_c = a_v[pl.ds(c * Rc, Rc), :].astype(jnp.bfloat16)
            part_c = jnp.dot(a_c, b_bf16, preferred_element_type=jnp.float32)
            xsend[pl.ds(c * Rc, Rc), :] = part_c.astype(jnp.bfloat16)
            if c == 0:
                pl.semaphore_wait(barrier_sem, 2)
            rdma = pltpu.make_async_remote_copy(
                src_ref=xsend.at[pl.ds(c * Rc, Rc), :],
                dst_ref=xrecv.at[pl.ds(c * Rc, Rc), :],
                send_sem=xsend_sems.at[c],
                recv_sem=xrecv_sems.at[c],
                device_id=peer_x,
                device_id_type=pl.DeviceIdType.MESH,
            )
            rdma.start()
            x_rdmas.append(rdma)

        y_rdmas = []
        out_cps = []
        for c in range(C):
            x_rdmas[c].wait_recv()
            ystage[pl.ds(c * Rc, Rc), :] = (
                xsend[pl.ds(c * Rc, Rc), :] + xrecv[pl.ds(c * Rc, Rc), :])
            rdma = pltpu.make_async_remote_copy(
                src_ref=ystage.at[pl.ds(c * Rc, Rc), :],
                dst_ref=yrecv.at[pl.ds(c * Rc, Rc), :],
                send_sem=ysend_sems.at[c],
                recv_sem=yrecv_sems.at[c],
                device_id=peer_y,
                device_id_type=pl.DeviceIdType.MESH,
            )
            rdma.start()
            y_rdmas.append(rdma)
            cp = pltpu.make_async_copy(
                ystage.at[pl.ds(c * Rc, Rc), :],
                out_hbm.at[pl.ds(my_base + c * Rc, Rc), :],
                out_sems.at[c])
            cp.start()
            out_cps.append(cp)

        for c in range(C):
            y_rdmas[c].wait_recv()
            cp = pltpu.make_async_copy(
                yrecv.at[pl.ds(c * Rc, Rc), :],
                out_hbm.at[pl.ds(other_base + c * Rc, Rc), :],
                out_sems.at[C + c])
            cp.start()
            out_cps.append(cp)

        for cp in out_cps:
            cp.wait()
        for c in range(C):
            x_rdmas[c].wait_send()
            y_rdmas[c].wait_send()

    return pl.pallas_call(
        body,
        out_shape=jax.ShapeDtypeStruct((M, N), jnp.bfloat16),
        in_specs=[
            pl.BlockSpec(memory_space=pl.ANY),
            pl.BlockSpec(memory_space=pl.ANY),
        ],
        out_specs=pl.BlockSpec(memory_space=pl.ANY),
        scratch_shapes=[
            pltpu.VMEM((Mh, K), jnp.float32),
            pltpu.VMEM((K, N), jnp.float32),
            pltpu.VMEM((Mh, N), jnp.bfloat16),
            pltpu.VMEM((Mh, N), jnp.bfloat16),
            pltpu.VMEM((Mh, N), jnp.bfloat16),
            pltpu.VMEM((Mh, N), jnp.bfloat16),
            pltpu.SemaphoreType.DMA((2,)),
            pltpu.SemaphoreType.DMA((2 * C,)),
            pltpu.SemaphoreType.DMA((C,)),
            pltpu.SemaphoreType.DMA((C,)),
            pltpu.SemaphoreType.DMA((C,)),
            pltpu.SemaphoreType.DMA((C,)),
        ],
        compiler_params=pltpu.CompilerParams(collective_id=0),
    )(A, B)
